# baseline (speedup 1.0000x reference)
import functools

import jax
import jax.numpy as jnp
import numpy as np
from jax import lax
from jax.experimental import pallas as pl
from jax.experimental.pallas import tpu as pltpu
from jax.experimental.pallas import tpu_sc as plsc

B = 4096
S = 200
D = 64
OUT = 64
V = 1_000_000
NC = 2
NS = 16
NW = NC * NS
BPW = B // NW
CH0 = 104
CH1 = S - CH0
NLANE = 16
NJ = OUT // NLANE
CONV_C = 16384
CONV_G = (V + CONV_C - 1) // CONV_C
CSH = CONV_C.bit_length() - 1
QSH = CSH - 2
QMASK = CONV_C // 4 - 1
TROWS = CONV_G * CONV_C
BLK_STARTS = tuple(k * NLANE for k in range(S // NLANE)) + (S - NLANE,)
_POS = (0, 32, 16, 48)


def _pooled_body(tok_hbm, table_hbm, b_hbm, out_hbm,
                 tok_v, b_v, row0_v, row1_v, rows_v, pooled_v, sem0, sem1):
    wid = lax.axis_index("s") * NC + lax.axis_index("c")
    base = wid * BPW
    pltpu.sync_copy(tok_hbm.at[pl.ds(base, BPW)], tok_v)
    pltpu.sync_copy(b_hbm, b_v)

    sems = (sem0, sem1)
    rowbufs = (row0_v, row1_v)

    def issue(i, nb):
        rv = rowbufs[nb]
        for st in BLK_STARTS:
            t = tok_v[i, pl.ds(st, NLANE)]
            rv[pl.ds(st, NLANE)] = (
                ((t >> CSH) << CSH) + ((t & QMASK) << 2) + ((t >> QSH) & 3))
        pltpu.async_copy(
            table_hbm.at[rv.at[pl.ds(0, CH0)]],
            rows_v.at[nb, pl.ds(0, CH0)], sems[nb])
        pltpu.async_copy(
            table_hbm.at[rv.at[pl.ds(CH0, CH1)]],
            rows_v.at[nb, pl.ds(CH0, CH1)], sems[nb])

    def drain(nb):
        rv = rowbufs[nb]
        pltpu.make_async_copy(
            table_hbm.at[rv.at[pl.ds(0, CH0)]],
            rows_v.at[nb, pl.ds(0, CH0)], sems[nb]).wait()
        pltpu.make_async_copy(
            table_hbm.at[rv.at[pl.ds(CH0, CH1)]],
            rows_v.at[nb, pl.ds(CH0, CH1)], sems[nb]).wait()

    issue(0, 0)
    issue(1, 1)

    bias = tuple(b_v[pl.ds(_POS[j], NLANE)] for j in range(NJ))

    def group_body(g, carry):
        for nb in range(2):
            i = g * 2 + nb
            drain(nb)

            def acc_body(s_, accs):
                new = []
                for h in range(2):
                    pk = rows_v[nb, s_, pl.ds(h * NLANE, NLANE)]
                    bf = plsc.bitcast(pk, jnp.bfloat16)
                    lo, hi = plsc.unpack(
                        bf, format=plsc.PackFormat.INTERLEAVED)
                    new.append(accs[2 * h] + lo)
                    new.append(accs[2 * h + 1] + hi)
                return tuple(new)

            accs = lax.fori_loop(
                0, S, acc_body,
                tuple(jnp.zeros((NLANE,), jnp.float32) for _ in range(NJ)),
            )

            @pl.when(i + 2 < BPW)
            def _():
                issue(i + 2, nb)

            for j in range(NJ):
                pooled_v[i, pl.ds(_POS[j], NLANE)] = (
                    accs[j] * (1.0 / S) + bias[j])
        return carry

    lax.fori_loop(0, BPW // 2, group_body, 0)
    pltpu.sync_copy(pooled_v, out_hbm.at[pl.ds(base, BPW)])


_pooled = functools.partial(
    pl.kernel,
    out_type=jax.ShapeDtypeStruct((B, OUT), jnp.float32),
    mesh=plsc.VectorSubcoreMesh(core_axis_name="c", subcore_axis_name="s"),
    scratch_types=[
        pltpu.VMEM((BPW, S), jnp.int32),
        pltpu.VMEM((OUT,), jnp.float32),
        pltpu.VMEM((S,), jnp.int32),
        pltpu.VMEM((S,), jnp.int32),
        pltpu.VMEM((2, S, D // 2), jnp.float32),
        pltpu.VMEM((BPW, OUT), jnp.float32),
        pltpu.SemaphoreType.DMA,
        pltpu.SemaphoreType.DMA,
    ],
    compiler_params=pltpu.CompilerParams(
        use_tc_tiling_on_sc=False, needs_layout_passes=False),
)(_pooled_body)


def _conv_body(xt_ref, w_ref, o_ref):
    res = lax.dot_general(
        xt_ref[...], w_ref[...],
        (((0,), (1,)), ((), ())),
        preferred_element_type=jnp.float32,
    )
    u = lax.bitcast_convert_type(res, jnp.uint32)
    r = (u + jnp.uint32(0x7FFF) + ((u >> 16) & jnp.uint32(1))) >> 16
    pku = r[:, 0:OUT // 2] | (r[:, OUT // 2:OUT] << 16)
    pk = lax.bitcast_convert_type(pku, jnp.float32)
    q = CONV_C // 4
    packed = jnp.concatenate(
        [pk[0:q], pk[q:2 * q], pk[2 * q:3 * q], pk[3 * q:4 * q]], axis=1)
    o_ref[...] = packed.reshape(CONV_C * (OUT // 2))


_convert = pl.pallas_call(
    _conv_body,
    out_shape=jax.ShapeDtypeStruct((TROWS * (OUT // 2),), jnp.float32),
    grid=(CONV_G,),
    in_specs=[
        pl.BlockSpec((D, CONV_C), lambda i: (0, i)),
        pl.BlockSpec((OUT, D), lambda i: (0, 0)),
    ],
    out_specs=pl.BlockSpec((CONV_C * (OUT // 2),), lambda i: (i,)),
)


def kernel(token_ids, table, W, b):
    tok = token_ids.astype(jnp.int32)
    tabw = _convert(table.T, W).reshape(TROWS, OUT // 2)
    return _pooled(tok, tabw, b)


# speedup vs baseline: 1.2807x; 1.2807x over previous
import functools

import jax
import jax.numpy as jnp
from jax import lax
from jax.experimental import pallas as pl
from jax.experimental.pallas import tpu as pltpu
from jax.experimental.pallas import tpu_sc as plsc

B = 4096
S = 200
D = 64
OUT = 64
V = 1_000_000
NC = 2
NS = 16
NW = NC * NS
BPW = B // NW
CH0 = 104
CH1 = S - CH0
NLANE = 16
NJ = D // NLANE
CONV_C = 32768
CONV_G = (V + CONV_C - 1) // CONV_C
CSH = CONV_C.bit_length() - 1
HSH = CSH - 1
HMASK = CONV_C // 2 - 1
TROWS = CONV_G * CONV_C
BLK_STARTS = tuple(k * NLANE for k in range(S // NLANE)) + (S - NLANE,)


def _pooled_body(tok_hbm, table_hbm, b_hbm, out_hbm,
                 tok_v, b_v, row0_v, row1_v, rows_v, pooled_v, sem0, sem1):
    wid = lax.axis_index("s") * NC + lax.axis_index("c")
    base = wid * BPW
    pltpu.sync_copy(tok_hbm.at[pl.ds(base, BPW)], tok_v)
    pltpu.sync_copy(b_hbm, b_v)

    sems = (sem0, sem1)
    rowbufs = (row0_v, row1_v)

    def issue(i, nb):
        rv = rowbufs[nb]
        for st in BLK_STARTS:
            t = tok_v[i, pl.ds(st, NLANE)]
            rv[pl.ds(st, NLANE)] = (
                ((t >> CSH) << CSH) + ((t & HMASK) << 1) + ((t >> HSH) & 1))
        pltpu.async_copy(
            table_hbm.at[rv.at[pl.ds(0, CH0)]],
            rows_v.at[nb, pl.ds(0, CH0)], sems[nb])
        pltpu.async_copy(
            table_hbm.at[rv.at[pl.ds(CH0, CH1)]],
            rows_v.at[nb, pl.ds(CH0, CH1)], sems[nb])

    def drain(nb):
        rv = rowbufs[nb]
        pltpu.make_async_copy(
            table_hbm.at[rv.at[pl.ds(0, CH0)]],
            rows_v.at[nb, pl.ds(0, CH0)], sems[nb]).wait()
        pltpu.make_async_copy(
            table_hbm.at[rv.at[pl.ds(CH0, CH1)]],
            rows_v.at[nb, pl.ds(CH0, CH1)], sems[nb]).wait()

    issue(0, 0)
    issue(1, 1)

    bias = tuple(b_v[pl.ds(j * NLANE, NLANE)] for j in range(NJ))

    def group_body(g, carry):
        for nb in range(2):
            i = g * 2 + nb
            drain(nb)

            def acc_body(s_, accs):
                return tuple(
                    accs[j] + rows_v[nb, s_, pl.ds(j * NLANE, NLANE)]
                    for j in range(NJ)
                )

            accs = lax.fori_loop(
                0, S, acc_body,
                tuple(jnp.zeros((NLANE,), jnp.float32) for _ in range(NJ)),
            )

            @pl.when(i + 2 < BPW)
            def _():
                issue(i + 2, nb)

            for j in range(NJ):
                pooled_v[i, pl.ds(j * NLANE, NLANE)] = (
                    accs[j] * (1.0 / S) + bias[j])
        return carry

    lax.fori_loop(0, BPW // 2, group_body, 0)
    pltpu.sync_copy(pooled_v, out_hbm.at[pl.ds(base, BPW)])


_pooled = functools.partial(
    pl.kernel,
    out_type=jax.ShapeDtypeStruct((B, OUT), jnp.float32),
    mesh=plsc.VectorSubcoreMesh(core_axis_name="c", subcore_axis_name="s"),
    scratch_types=[
        pltpu.VMEM((BPW, S), jnp.int32),
        pltpu.VMEM((D,), jnp.float32),
        pltpu.VMEM((S,), jnp.int32),
        pltpu.VMEM((S,), jnp.int32),
        pltpu.VMEM((2, S, D), jnp.float32),
        pltpu.VMEM((BPW, OUT), jnp.float32),
        pltpu.SemaphoreType.DMA,
        pltpu.SemaphoreType.DMA,
    ],
    compiler_params=pltpu.CompilerParams(use_tc_tiling_on_sc=False),
)(_pooled_body)


def _conv_body(xt_ref, w_ref, o_ref):
    res = lax.dot_general(
        xt_ref[...], w_ref[...],
        (((0,), (1,)), ((), ())),
        preferred_element_type=jnp.float32,
    )
    packed = jnp.concatenate(
        [res[0:CONV_C // 2], res[CONV_C // 2:CONV_C]], axis=1)
    o_ref[...] = packed.reshape(CONV_C * OUT)


_convert = pl.pallas_call(
    _conv_body,
    out_shape=jax.ShapeDtypeStruct((TROWS * OUT,), jnp.float32),
    grid=(CONV_G,),
    in_specs=[
        pl.BlockSpec((D, CONV_C), lambda i: (0, i)),
        pl.BlockSpec((OUT, D), lambda i: (0, 0)),
    ],
    out_specs=pl.BlockSpec((CONV_C * OUT,), lambda i: (i,)),
)


def kernel(token_ids, table, W, b):
    tok = token_ids.astype(jnp.int32)
    tabw = _convert(table.T, W).reshape(TROWS, OUT)
    return _pooled(tok, tabw, b)


# speedup vs baseline: 1.3287x; 1.0375x over previous
import functools

import jax
import jax.numpy as jnp
from jax import lax
from jax.experimental import pallas as pl
from jax.experimental.pallas import tpu as pltpu
from jax.experimental.pallas import tpu_sc as plsc

B = 4096
S = 200
D = 64
OUT = 64
V = 1_000_000
NC = 2
NS = 16
NW = NC * NS
BPW = B // NW
CH0 = 104
CH1 = S - CH0
NLANE = 16
NJ = D // NLANE
CONV_C = 32768
CONV_G = (V + CONV_C - 1) // CONV_C
CSH = CONV_C.bit_length() - 1
HSH = CSH - 1
HMASK = CONV_C // 2 - 1
TROWS = CONV_G * CONV_C
BLK_STARTS = tuple(k * NLANE for k in range(S // NLANE)) + (S - NLANE,)


def _pooled_body(tok_hbm, table_hbm, b_hbm, out_hbm,
                 tok_v, b_v, row0_v, row1_v, rows_v, pooled_v, sem0, sem1):
    wid = lax.axis_index("s") * NC + lax.axis_index("c")
    base = wid * BPW
    pltpu.sync_copy(tok_hbm.at[pl.ds(base, BPW)], tok_v)
    pltpu.sync_copy(b_hbm, b_v)

    sems = (sem0, sem1)
    rowbufs = (row0_v, row1_v)

    def issue(i, nb):
        rv = rowbufs[nb]
        for st in BLK_STARTS:
            t = tok_v[i, pl.ds(st, NLANE)]
            rv[pl.ds(st, NLANE)] = (
                ((t >> CSH) << CSH) + ((t & HMASK) << 1) + ((t >> HSH) & 1))
        pltpu.async_copy(
            table_hbm.at[rv.at[pl.ds(0, CH0)]],
            rows_v.at[nb, pl.ds(0, CH0)], sems[nb])
        pltpu.async_copy(
            table_hbm.at[rv.at[pl.ds(CH0, CH1)]],
            rows_v.at[nb, pl.ds(CH0, CH1)], sems[nb])

    def drain(nb):
        rv = rowbufs[nb]
        pltpu.make_async_copy(
            table_hbm.at[rv.at[pl.ds(0, CH0)]],
            rows_v.at[nb, pl.ds(0, CH0)], sems[nb]).wait()
        pltpu.make_async_copy(
            table_hbm.at[rv.at[pl.ds(CH0, CH1)]],
            rows_v.at[nb, pl.ds(CH0, CH1)], sems[nb]).wait()

    issue(0, 0)
    issue(1, 1)

    bias = tuple(b_v[pl.ds(j * NLANE, NLANE)] for j in range(NJ))

    def group_body(g, carry):
        for nb in range(2):
            i = g * 2 + nb
            drain(nb)

            def acc_body(t8, accs):
                for u in range(8):
                    s_ = t8 * 8 + u
                    accs = tuple(
                        accs[j] + rows_v[nb, s_, pl.ds(j * NLANE, NLANE)]
                        for j in range(NJ)
                    )
                return accs

            accs = lax.fori_loop(
                0, S // 8, acc_body,
                tuple(jnp.zeros((NLANE,), jnp.float32) for _ in range(NJ)),
            )

            @pl.when(i + 2 < BPW)
            def _():
                issue(i + 2, nb)

            for j in range(NJ):
                pooled_v[i, pl.ds(j * NLANE, NLANE)] = (
                    accs[j] * (1.0 / S) + bias[j])
        return carry

    lax.fori_loop(0, BPW // 2, group_body, 0)
    pltpu.sync_copy(pooled_v, out_hbm.at[pl.ds(base, BPW)])


_pooled = functools.partial(
    pl.kernel,
    out_type=jax.ShapeDtypeStruct((B, OUT), jnp.float32),
    mesh=plsc.VectorSubcoreMesh(core_axis_name="c", subcore_axis_name="s"),
    scratch_types=[
        pltpu.VMEM((BPW, S), jnp.int32),
        pltpu.VMEM((D,), jnp.float32),
        pltpu.VMEM((S,), jnp.int32),
        pltpu.VMEM((S,), jnp.int32),
        pltpu.VMEM((2, S, D), jnp.float32),
        pltpu.VMEM((BPW, OUT), jnp.float32),
        pltpu.SemaphoreType.DMA,
        pltpu.SemaphoreType.DMA,
    ],
    compiler_params=pltpu.CompilerParams(use_tc_tiling_on_sc=False),
)(_pooled_body)


def _conv_body(xt_ref, w_ref, o_ref):
    res = lax.dot_general(
        xt_ref[...], w_ref[...],
        (((0,), (1,)), ((), ())),
        preferred_element_type=jnp.float32,
    )
    packed = jnp.concatenate(
        [res[0:CONV_C // 2], res[CONV_C // 2:CONV_C]], axis=1)
    o_ref[...] = packed.reshape(CONV_C * OUT)


_convert = pl.pallas_call(
    _conv_body,
    out_shape=jax.ShapeDtypeStruct((TROWS * OUT,), jnp.float32),
    grid=(CONV_G,),
    in_specs=[
        pl.BlockSpec((D, CONV_C), lambda i: (0, i)),
        pl.BlockSpec((OUT, D), lambda i: (0, 0)),
    ],
    out_specs=pl.BlockSpec((CONV_C * OUT,), lambda i: (i,)),
)


def kernel(token_ids, table, W, b):
    tok = token_ids.astype(jnp.int32)
    tabw = _convert(table.T, W).reshape(TROWS, OUT)
    return _pooled(tok, tabw, b)


# speedup vs baseline: 1.4884x; 1.1202x over previous
import functools

import jax
import jax.numpy as jnp
from jax import lax
from jax.experimental import pallas as pl
from jax.experimental.pallas import tpu as pltpu
from jax.experimental.pallas import tpu_sc as plsc

B = 4096
S = 200
D = 64
OUT = 64
V = 1_000_000
NC = 2
NS = 16
NW = NC * NS
BPW = B // NW
CH0 = 104
CH1 = S - CH0
NLANE = 16
NJ = D // NLANE
CONV_C = 32768
CONV_G = (V + CONV_C - 1) // CONV_C
CSH = CONV_C.bit_length() - 1
HSH = CSH - 1
HMASK = CONV_C // 2 - 1
TROWS = CONV_G * CONV_C
BLK_STARTS = tuple(k * NLANE for k in range(S // NLANE)) + (S - NLANE,)


def _pooled_body(tok_hbm, table_hbm, b_hbm, out_hbm,
                 tok_v, b_v, row0_v, row1_v, row2_v, row3_v, rows_v, pooled_v,
                 sem0, sem1, sem2, sem3):
    wid = lax.axis_index("s") * NC + lax.axis_index("c")
    base = wid * BPW
    pltpu.sync_copy(tok_hbm.at[pl.ds(base, BPW)], tok_v)
    pltpu.sync_copy(b_hbm, b_v)

    sems = (sem0, sem1, sem2, sem3)
    rowbufs = (row0_v, row1_v, row2_v, row3_v)

    def issue(i, nb):
        rv = rowbufs[nb]
        for st in BLK_STARTS:
            t = tok_v[i, pl.ds(st, NLANE)]
            rv[pl.ds(st, NLANE)] = (
                ((t >> CSH) << CSH) + ((t & HMASK) << 1) + ((t >> HSH) & 1))
        pltpu.async_copy(
            table_hbm.at[rv.at[pl.ds(0, CH0)]],
            rows_v.at[nb, pl.ds(0, CH0)], sems[nb])
        pltpu.async_copy(
            table_hbm.at[rv.at[pl.ds(CH0, CH1)]],
            rows_v.at[nb, pl.ds(CH0, CH1)], sems[nb])

    def drain(nb):
        rv = rowbufs[nb]
        pltpu.make_async_copy(
            table_hbm.at[rv.at[pl.ds(0, CH0)]],
            rows_v.at[nb, pl.ds(0, CH0)], sems[nb]).wait()
        pltpu.make_async_copy(
            table_hbm.at[rv.at[pl.ds(CH0, CH1)]],
            rows_v.at[nb, pl.ds(CH0, CH1)], sems[nb]).wait()

    for _i in range(4):
        issue(_i, _i)

    bias = tuple(b_v[pl.ds(j * NLANE, NLANE)] for j in range(NJ))

    def group_body(g, carry):
        for nb in range(4):
            i = g * 4 + nb
            drain(nb)

            def acc_body(t8, accs):
                for u in range(8):
                    s_ = t8 * 8 + u
                    accs = tuple(
                        accs[j] + rows_v[nb, s_, pl.ds(j * NLANE, NLANE)]
                        for j in range(NJ)
                    )
                return accs

            accs = lax.fori_loop(
                0, S // 8, acc_body,
                tuple(jnp.zeros((NLANE,), jnp.float32) for _ in range(NJ)),
            )

            @pl.when(i + 4 < BPW)
            def _():
                issue(i + 4, nb)

            for j in range(NJ):
                pooled_v[i, pl.ds(j * NLANE, NLANE)] = (
                    accs[j] * (1.0 / S) + bias[j])
        return carry

    lax.fori_loop(0, BPW // 4, group_body, 0)
    pltpu.sync_copy(pooled_v, out_hbm.at[pl.ds(base, BPW)])


_pooled = functools.partial(
    pl.kernel,
    out_type=jax.ShapeDtypeStruct((B, OUT), jnp.float32),
    mesh=plsc.VectorSubcoreMesh(core_axis_name="c", subcore_axis_name="s"),
    scratch_types=[
        pltpu.VMEM((BPW, S), jnp.int32),
        pltpu.VMEM((D,), jnp.float32),
        pltpu.VMEM((S,), jnp.int32),
        pltpu.VMEM((S,), jnp.int32),
        pltpu.VMEM((S,), jnp.int32),
        pltpu.VMEM((S,), jnp.int32),
        pltpu.VMEM((4, S, D), jnp.float32),
        pltpu.VMEM((BPW, OUT), jnp.float32),
        pltpu.SemaphoreType.DMA,
        pltpu.SemaphoreType.DMA,
        pltpu.SemaphoreType.DMA,
        pltpu.SemaphoreType.DMA,
    ],
    compiler_params=pltpu.CompilerParams(use_tc_tiling_on_sc=False),
)(_pooled_body)


def _conv_body(xt_ref, w_ref, o_ref):
    res = lax.dot_general(
        xt_ref[...], w_ref[...],
        (((0,), (1,)), ((), ())),
        preferred_element_type=jnp.float32,
    )
    packed = jnp.concatenate(
        [res[0:CONV_C // 2], res[CONV_C // 2:CONV_C]], axis=1)
    o_ref[...] = packed.reshape(CONV_C * OUT)


_convert = pl.pallas_call(
    _conv_body,
    out_shape=jax.ShapeDtypeStruct((TROWS * OUT,), jnp.float32),
    grid=(CONV_G,),
    in_specs=[
        pl.BlockSpec((D, CONV_C), lambda i: (0, i)),
        pl.BlockSpec((OUT, D), lambda i: (0, 0)),
    ],
    out_specs=pl.BlockSpec((CONV_C * OUT,), lambda i: (i,)),
)


def kernel(token_ids, table, W, b):
    tok = token_ids.astype(jnp.int32)
    tabw = _convert(table.T, W).reshape(TROWS, OUT)
    return _pooled(tok, tabw, b)
